# baseline (speedup 1.0000x reference)
import functools

import jax
import jax.numpy as jnp
from jax import lax
from jax.experimental import pallas as pl
from jax.experimental.pallas import tpu as pltpu
from jax.experimental.pallas import tpu_sc as plsc

VOCAB = 100000
EMB = 16
BATCH = 1024
KDIM = 24
VT = 1024
NBLOCKS = 98
VLAST = VOCAB - VT * (NBLOCKS - 1)


_NC, _NS = 2, 16
_NW = _NC * _NS
_B_PER_W = BATCH // _NW


def _sc_gather(inputs, embeddings):
    mesh = plsc.VectorSubcoreMesh(core_axis_name="c", subcore_axis_name="s")

    @functools.partial(
        pl.kernel,
        mesh=mesh,
        out_type=jax.ShapeDtypeStruct((BATCH, EMB), jnp.float32),
        scratch_types=[
            pltpu.VMEM((_B_PER_W,), jnp.int32),
            pltpu.VMEM((_B_PER_W, EMB), jnp.float32),
            pltpu.SemaphoreType.DMA,
        ],
        compiler_params=pltpu.CompilerParams(use_tc_tiling_on_sc=False),
    )
    def gather_kernel(idx_hbm, table_hbm, out_hbm, idx_v, rows_v, sem):
        wid = lax.axis_index("s") * _NC + lax.axis_index("c")
        base = wid * _B_PER_W
        pltpu.sync_copy(idx_hbm.at[pl.ds(base, _B_PER_W)], idx_v)
        pltpu.async_copy(table_hbm.at[idx_v], rows_v, sem).wait()
        pltpu.sync_copy(rows_v, out_hbm.at[pl.ds(base, _B_PER_W)])

    return gather_kernel(inputs, embeddings)


_NBUF = 2
_NSTEPS = NBLOCKS // _NBUF
_WSTEP = VT * _NBUF


def _blk_copy(bufs, out_ref, sem, blk, j, rows=VT):
    return pltpu.make_async_copy(
        bufs[j].at[pl.ds(0, rows), :],
        out_ref.at[pl.ds(blk * VT, rows), :],
        sem.at[j],
    )


def _proj_body(emb_ref, wt_ref, b_ref, out_ref, *scratch):
    bufs = scratch[:_NBUF]
    embt = scratch[_NBUF]
    wtb = scratch[_NBUF + 1]
    sem = scratch[_NBUF + 2]
    i = pl.program_id(0)

    @pl.when(i == 0)
    def _prep():
        embt[0:EMB, :] = emb_ref[...].T
        embt[EMB : EMB + 1, :] = jnp.ones((1, BATCH), jnp.float32)
        embt[EMB + 1 : KDIM, :] = jnp.zeros((KDIM - EMB - 1, BATCH), jnp.float32)
        wtb[EMB + 1 : KDIM, :] = jnp.zeros((KDIM - EMB - 1, VT), jnp.float32)

    for j in range(_NBUF):
        blk = i * _NBUF + j

        @pl.when(i > 0)
        def _wait_prev():
            prev = (i - 1) * _NBUF + j
            pltpu.make_async_copy(
                bufs[j], out_ref.at[pl.ds(prev * VT, VT), :], sem.at[j]
            ).wait()

        wtb[0:EMB, :] = wt_ref[:, pl.ds(j * VT, VT)]
        wtb[EMB : EMB + 1, :] = b_ref[:, pl.ds(j * VT, VT)]
        bufs[j][...] = lax.dot_general(
            wtb[...],
            embt[...],
            (((0,), (0,)), ((), ())),
            preferred_element_type=jnp.float32,
        )
        if j == _NBUF - 1:
            @pl.when(i < _NSTEPS - 1)
            def _full():
                _blk_copy(bufs, out_ref, sem, blk, j).start()

            @pl.when(i == _NSTEPS - 1)
            def _partial():
                _blk_copy(bufs, out_ref, sem, blk, j, VLAST).start()

        else:
            _blk_copy(bufs, out_ref, sem, blk, j).start()

    @pl.when(i == _NSTEPS - 1)
    def _drain():
        for j in range(_NBUF):
            blk = i * _NBUF + j
            rows = VLAST if j == _NBUF - 1 else VT
            _blk_copy(bufs, out_ref, sem, blk, j, rows).wait()


def _tc_project_t(emb, Wt, b2d):
    return pl.pallas_call(
        _proj_body,
        grid=(_NSTEPS,),
        in_specs=[
            pl.BlockSpec(memory_space=pltpu.VMEM),
            pl.BlockSpec((EMB, _WSTEP), lambda i: (0, i)),
            pl.BlockSpec((1, _WSTEP), lambda i: (0, i)),
        ],
        out_specs=pl.BlockSpec(memory_space=pl.ANY),
        out_shape=jax.ShapeDtypeStruct((VOCAB, BATCH), jnp.float32),
        scratch_shapes=[pltpu.VMEM((VT, BATCH), jnp.float32) for _ in range(_NBUF)]
        + [
            pltpu.VMEM((KDIM, BATCH), jnp.float32),
            pltpu.VMEM((KDIM, VT), jnp.float32),
            pltpu.SemaphoreType.DMA((_NBUF,)),
        ],
        compiler_params=pltpu.CompilerParams(
            dimension_semantics=("arbitrary",),
        ),
    )(emb, Wt, b2d)


@jax.jit
def kernel(inputs, embeddings, W, b):
    emb = _sc_gather(inputs, embeddings)
    return _tc_project_t(emb, W.T, b.reshape(1, VOCAB)).T


# speedup vs baseline: 1.2772x; 1.2772x over previous
import functools

import jax
import jax.numpy as jnp
from jax import lax
from jax.experimental import pallas as pl
from jax.experimental.pallas import tpu as pltpu
from jax.experimental.pallas import tpu_sc as plsc

VOCAB = 100000
EMB = 16
BATCH = 1024
KDIM = 24
VT = 1024
NBLOCKS = 98
VLAST = VOCAB - VT * (NBLOCKS - 1)


_NC, _NS = 2, 16
_NW = _NC * _NS
_B_PER_W = BATCH // _NW


def _sc_gather_t(inputs, table_flat):
    mesh = plsc.VectorSubcoreMesh(core_axis_name="c", subcore_axis_name="s")

    @functools.partial(
        pl.kernel,
        mesh=mesh,
        out_type=jax.ShapeDtypeStruct((EMB, BATCH), jnp.float32),
        scratch_types=[
            pltpu.VMEM((_B_PER_W,), jnp.int32),
            pltpu.VMEM((EMB, _B_PER_W), jnp.int32),
            pltpu.VMEM((EMB, _B_PER_W), jnp.float32),
            pltpu.SemaphoreType.DMA,
        ],
        compiler_params=pltpu.CompilerParams(use_tc_tiling_on_sc=False),
    )
    def gather_kernel(idx_hbm, table_hbm, out_hbm, idx_v, fidx, vals, sem):
        wid = lax.axis_index("s") * _NC + lax.axis_index("c")
        base = wid * _B_PER_W
        pltpu.sync_copy(idx_hbm.at[pl.ds(base, _B_PER_W)], idx_v)
        for h in range(_B_PER_W // 16):
            chunk = idx_v[pl.ds(h * 16, 16)]
            for d in range(EMB):
                fidx[d, pl.ds(h * 16, 16)] = chunk + d * VOCAB
        copies = [
            pltpu.make_async_copy(table_hbm.at[fidx.at[d]], vals.at[d], sem)
            for d in range(EMB)
        ]
        for c in copies:
            c.start()
        for c in copies:
            c.wait()
        pltpu.sync_copy(vals, out_hbm.at[:, pl.ds(base, _B_PER_W)])

    return gather_kernel(inputs, table_flat)


_NBUF = 7
_NSTEPS = NBLOCKS // _NBUF
_WSTEP = VT * _NBUF


def _blk_copy(bufs, out_ref, sem, blk, j, rows=VT):
    return pltpu.make_async_copy(
        bufs[j].at[pl.ds(0, rows), :],
        out_ref.at[pl.ds(blk * VT, rows), :],
        sem.at[j],
    )


def _proj_body(emb_ref, wt_ref, b_ref, out_ref, *scratch):
    bufs = scratch[:_NBUF]
    embt = scratch[_NBUF]
    wtb = scratch[_NBUF + 1]
    sem = scratch[_NBUF + 2]
    i = pl.program_id(0)

    @pl.when(i == 0)
    def _prep():
        embt[0:EMB, :] = emb_ref[...]
        embt[EMB : EMB + 1, :] = jnp.ones((1, BATCH), jnp.float32)
        embt[EMB + 1 : KDIM, :] = jnp.zeros((KDIM - EMB - 1, BATCH), jnp.float32)
        wtb[EMB + 1 : KDIM, :] = jnp.zeros((KDIM - EMB - 1, VT), jnp.float32)

    for j in range(_NBUF):
        blk = i * _NBUF + j

        @pl.when(i > 0)
        def _wait_prev():
            prev = (i - 1) * _NBUF + j
            pltpu.make_async_copy(
                bufs[j], out_ref.at[pl.ds(prev * VT, VT), :], sem.at[j]
            ).wait()

        wtb[0:EMB, :] = wt_ref[:, pl.ds(j * VT, VT)]
        wtb[EMB : EMB + 1, :] = b_ref[:, pl.ds(j * VT, VT)]
        bufs[j][...] = lax.dot_general(
            wtb[...],
            embt[...],
            (((0,), (0,)), ((), ())),
            preferred_element_type=jnp.float32,
        )
        if j == _NBUF - 1:
            @pl.when(i < _NSTEPS - 1)
            def _full():
                _blk_copy(bufs, out_ref, sem, blk, j).start()

            @pl.when(i == _NSTEPS - 1)
            def _partial():
                _blk_copy(bufs, out_ref, sem, blk, j, VLAST).start()

        else:
            _blk_copy(bufs, out_ref, sem, blk, j).start()

    @pl.when(i == _NSTEPS - 1)
    def _drain():
        for j in range(_NBUF):
            blk = i * _NBUF + j
            rows = VLAST if j == _NBUF - 1 else VT
            _blk_copy(bufs, out_ref, sem, blk, j, rows).wait()


def _tc_project_t(emb, Wt, b2d):
    return pl.pallas_call(
        _proj_body,
        grid=(_NSTEPS,),
        in_specs=[
            pl.BlockSpec(memory_space=pltpu.VMEM),
            pl.BlockSpec((EMB, _WSTEP), lambda i: (0, i)),
            pl.BlockSpec((1, _WSTEP), lambda i: (0, i)),
        ],
        out_specs=pl.BlockSpec(memory_space=pl.ANY),
        out_shape=jax.ShapeDtypeStruct((VOCAB, BATCH), jnp.float32),
        scratch_shapes=[pltpu.VMEM((VT, BATCH), jnp.float32) for _ in range(_NBUF)]
        + [
            pltpu.VMEM((KDIM, BATCH), jnp.float32),
            pltpu.VMEM((KDIM, VT), jnp.float32),
            pltpu.SemaphoreType.DMA((_NBUF,)),
        ],
        compiler_params=pltpu.CompilerParams(
            dimension_semantics=("arbitrary",),
        ),
    )(emb, Wt, b2d)


@jax.jit
def kernel(inputs, embeddings, W, b):
    emb_t = _sc_gather_t(inputs, embeddings.T.reshape(-1))
    return _tc_project_t(emb_t, W.T, b.reshape(1, VOCAB)).T


# speedup vs baseline: 1.2812x; 1.0031x over previous
import functools

import jax
import jax.numpy as jnp
from jax import lax
from jax.experimental import pallas as pl
from jax.experimental.pallas import tpu as pltpu
from jax.experimental.pallas import tpu_sc as plsc

VOCAB = 100000
EMB = 16
BATCH = 1024
KDIM = 24
VT = 512
NBLOCKS = 196
VLAST = VOCAB - VT * (NBLOCKS - 1)


_NC, _NS = 2, 16
_NW = _NC * _NS
_B_PER_W = BATCH // _NW


def _sc_gather_t(inputs, table_flat):
    mesh = plsc.VectorSubcoreMesh(core_axis_name="c", subcore_axis_name="s")

    @functools.partial(
        pl.kernel,
        mesh=mesh,
        out_type=jax.ShapeDtypeStruct((EMB, BATCH), jnp.float32),
        scratch_types=[
            pltpu.VMEM((_B_PER_W,), jnp.int32),
            pltpu.VMEM((EMB, _B_PER_W), jnp.int32),
            pltpu.VMEM((EMB, _B_PER_W), jnp.float32),
            pltpu.SemaphoreType.DMA,
        ],
        compiler_params=pltpu.CompilerParams(use_tc_tiling_on_sc=False),
    )
    def gather_kernel(idx_hbm, table_hbm, out_hbm, idx_v, fidx, vals, sem):
        wid = lax.axis_index("s") * _NC + lax.axis_index("c")
        base = wid * _B_PER_W
        pltpu.sync_copy(idx_hbm.at[pl.ds(base, _B_PER_W)], idx_v)
        for h in range(_B_PER_W // 16):
            chunk = idx_v[pl.ds(h * 16, 16)]
            for d in range(EMB):
                fidx[d, pl.ds(h * 16, 16)] = chunk + d * VOCAB
        copies = [
            pltpu.make_async_copy(table_hbm.at[fidx.at[d]], vals.at[d], sem)
            for d in range(EMB)
        ]
        for c in copies:
            c.start()
        for c in copies:
            c.wait()
        pltpu.sync_copy(vals, out_hbm.at[:, pl.ds(base, _B_PER_W)])

    return gather_kernel(inputs, table_flat)


_NBUF = 14
_NSTEPS = NBLOCKS // _NBUF
_WSTEP = VT * _NBUF


def _blk_copy(bufs, out_ref, sem, blk, j, rows=VT):
    return pltpu.make_async_copy(
        bufs[j].at[pl.ds(0, rows), :],
        out_ref.at[pl.ds(blk * VT, rows), :],
        sem.at[j],
    )


def _proj_body(emb_ref, wt_ref, b_ref, out_ref, *scratch):
    bufs = scratch[:_NBUF]
    embt = scratch[_NBUF]
    wtb = scratch[_NBUF + 1]
    sem = scratch[_NBUF + 2]
    i = pl.program_id(0)

    @pl.when(i == 0)
    def _prep():
        embt[0:EMB, :] = emb_ref[...]
        embt[EMB : EMB + 1, :] = jnp.ones((1, BATCH), jnp.float32)
        embt[EMB + 1 : KDIM, :] = jnp.zeros((KDIM - EMB - 1, BATCH), jnp.float32)
        wtb[EMB + 1 : KDIM, :] = jnp.zeros((KDIM - EMB - 1, VT), jnp.float32)

    for j in range(_NBUF):
        blk = i * _NBUF + j

        @pl.when(i > 0)
        def _wait_prev():
            prev = (i - 1) * _NBUF + j
            pltpu.make_async_copy(
                bufs[j], out_ref.at[pl.ds(prev * VT, VT), :], sem.at[j]
            ).wait()

        wtb[0:EMB, :] = wt_ref[:, pl.ds(j * VT, VT)]
        wtb[EMB : EMB + 1, :] = b_ref[:, pl.ds(j * VT, VT)]
        bufs[j][...] = lax.dot_general(
            wtb[...],
            embt[...],
            (((0,), (0,)), ((), ())),
            preferred_element_type=jnp.float32,
        )
        if j == _NBUF - 1:
            @pl.when(i < _NSTEPS - 1)
            def _full():
                _blk_copy(bufs, out_ref, sem, blk, j).start()

            @pl.when(i == _NSTEPS - 1)
            def _partial():
                _blk_copy(bufs, out_ref, sem, blk, j, VLAST).start()

        else:
            _blk_copy(bufs, out_ref, sem, blk, j).start()

    @pl.when(i == _NSTEPS - 1)
    def _drain():
        for j in range(_NBUF):
            blk = i * _NBUF + j
            rows = VLAST if j == _NBUF - 1 else VT
            _blk_copy(bufs, out_ref, sem, blk, j, rows).wait()


def _tc_project_t(emb, Wt, b2d):
    return pl.pallas_call(
        _proj_body,
        grid=(_NSTEPS,),
        in_specs=[
            pl.BlockSpec(memory_space=pltpu.VMEM),
            pl.BlockSpec((EMB, _WSTEP), lambda i: (0, i)),
            pl.BlockSpec((1, _WSTEP), lambda i: (0, i)),
        ],
        out_specs=pl.BlockSpec(memory_space=pl.ANY),
        out_shape=jax.ShapeDtypeStruct((VOCAB, BATCH), jnp.float32),
        scratch_shapes=[pltpu.VMEM((VT, BATCH), jnp.float32) for _ in range(_NBUF)]
        + [
            pltpu.VMEM((KDIM, BATCH), jnp.float32),
            pltpu.VMEM((KDIM, VT), jnp.float32),
            pltpu.SemaphoreType.DMA((_NBUF,)),
        ],
        compiler_params=pltpu.CompilerParams(
            dimension_semantics=("arbitrary",),
        ),
    )(emb, Wt, b2d)


@jax.jit
def kernel(inputs, embeddings, W, b):
    emb_t = _sc_gather_t(inputs, embeddings.T.reshape(-1))
    return _tc_project_t(emb_t, W.T, b.reshape(1, VOCAB)).T
